# baseline (speedup 1.0000x reference)
import functools

import jax
import jax.numpy as jnp
from jax import lax
from jax.experimental import pallas as pl
from jax.experimental.pallas import tpu as pltpu
from jax.experimental.pallas import tpu_sc as plsc

N = 10000
E = 320000
NC = 2
NS = 16
NW = NC * NS
K = 80
NCH = 125
NBUF = 5
GA = 3
D1 = 64
D2 = 16

_mesh = plsc.VectorSubcoreMesh(core_axis_name="c", subcore_axis_name="s")


NP = 10240


def _deg_call(dst3):
    @functools.partial(
        pl.kernel,
        out_type=jax.ShapeDtypeStruct((NC, NP), jnp.float32),
        mesh=_mesh,
        scratch_types=[
            pltpu.VMEM((NCH, K), jnp.int32),
            pltpu.VMEM((K,), jnp.float32),
            pltpu.VMEM((640,), jnp.float32),
            pltpu.VMEM_SHARED((NP,), jnp.float32),
            pltpu.SemaphoreType.DMA,
        ],
        compiler_params=pltpu.CompilerParams(use_tc_tiling_on_sc=False),
    )
    def k(dst_hbm, out_hbm, idx_v, ones_v, blk_v, acc_s, sem):
        c = lax.axis_index("c")
        s = lax.axis_index("s")
        wid = c * NS + s
        one16 = jnp.ones((16,), jnp.float32)
        zero16 = jnp.zeros((16,), jnp.float32)
        for i in range(K // 16):
            ones_v[pl.ds(i * 16, 16)] = one16
        for i in range(640 // 16):
            blk_v[pl.ds(i * 16, 16)] = zero16
        pltpu.sync_copy(blk_v, acc_s.at[pl.ds(s * 640, 640)])
        plsc.subcore_barrier()
        pltpu.sync_copy(dst_hbm.at[wid], idx_v)

        W = 8

        def body(j, carry):
            pltpu.async_copy(ones_v, acc_s.at[idx_v.at[j]], sem, add=True)

            @pl.when(j >= W)
            def _():
                pltpu.make_async_copy(ones_v, acc_s.at[idx_v.at[j - W]], sem).wait()

            return carry

        lax.fori_loop(0, NCH, body, 0)
        for t in range(W):
            pltpu.make_async_copy(ones_v, acc_s.at[idx_v.at[NCH - W + t]], sem).wait()
        plsc.subcore_barrier()
        pltpu.sync_copy(acc_s.at[pl.ds(s * 640, 640)], blk_v)
        pltpu.sync_copy(blk_v, out_hbm.at[c, pl.ds(s * 640, 640)])

    return k(dst3)


def _agg_call(y, src3, dst3, dp):
    @functools.partial(
        pl.kernel,
        out_type=jax.ShapeDtypeStruct((NC, NP, dp), jnp.float32),
        mesh=_mesh,
        scratch_types=[
            pltpu.VMEM((NCH, K), jnp.int32),
            pltpu.VMEM((NCH, K), jnp.int32),
            [pltpu.VMEM((K, dp), jnp.float32) for _ in range(NBUF)],
            pltpu.VMEM_SHARED((NP, dp), jnp.float32),
            [pltpu.SemaphoreType.DMA for _ in range(NBUF)],
            [pltpu.SemaphoreType.DMA for _ in range(NBUF)],
        ],
        compiler_params=pltpu.CompilerParams(use_tc_tiling_on_sc=False),
    )
    def k(y_hbm, src_hbm, dst_hbm, out_hbm, si_v, di_v, rows, acc_s, gsem, ssem):
        blk_v = rows[0]
        c = lax.axis_index("c")
        s = lax.axis_index("s")
        wid = c * NS + s
        zero16 = jnp.zeros((16,), jnp.float32)

        def zbody(j, carry):
            for t in range(dp // 16):
                blk_v[j, pl.ds(t * 16, 16)] = zero16
            return carry

        lax.fori_loop(0, K, zbody, 0)
        r0 = s * 640
        for t in range(640 // K):
            pltpu.sync_copy(blk_v, acc_s.at[pl.ds(r0 + t * K, K)])
        plsc.subcore_barrier()

        pltpu.sync_copy(src_hbm.at[wid], si_v)
        pltpu.sync_copy(dst_hbm.at[wid], di_v)

        def gstart(j, b):
            pltpu.async_copy(y_hbm.at[si_v.at[j]], rows[b], gsem[b])

        def gwait(j, b):
            pltpu.make_async_copy(y_hbm.at[si_v.at[j]], rows[b], gsem[b]).wait()

        def sstart(j, b):
            pltpu.async_copy(rows[b], acc_s.at[di_v.at[j]], ssem[b], add=True)

        def swait(j, b):
            pltpu.make_async_copy(rows[b], acc_s.at[di_v.at[j]], ssem[b]).wait()

        NR = NCH // NBUF
        for b in range(GA):
            gstart(b, b)

        def round_body(g, carry):
            j0 = g * NBUF
            for b in range(NBUF):
                j = j0 + b
                bp = (b + GA) % NBUF
                if b < NBUF - GA:
                    @pl.when(g >= 1)
                    def _():
                        swait(j + GA - NBUF, bp)

                    gstart(j + GA, bp)
                else:
                    @pl.when(g < NR - 1)
                    def _():
                        swait(j + GA - NBUF, bp)
                        gstart(j + GA, bp)

                gwait(j, b)
                sstart(j, b)
            return carry

        lax.fori_loop(0, NR, round_body, 0)
        for j in range(NCH - NBUF, NCH):
            swait(j, j % NBUF)
        plsc.subcore_barrier()

        for t in range(640 // K):
            pltpu.sync_copy(acc_s.at[pl.ds(r0 + t * K, K)], blk_v)
            pltpu.sync_copy(blk_v, out_hbm.at[c, pl.ds(r0 + t * K, K)])

    return k(y, src3, dst3)


BR = 1000


def _mm1_body(x_ref, w_ref, d0_ref, d1_ref, y_ref, dinv_ref):
    dinv = lax.rsqrt(d0_ref[...] + d1_ref[...] + 1.0)
    xw = jnp.dot(x_ref[...], w_ref[...], preferred_element_type=jnp.float32)
    y_ref[...] = xw * dinv
    dinv_ref[...] = dinv


def _mm1_call(x, w1p, d0, d1):
    return pl.pallas_call(
        _mm1_body,
        grid=(N // BR,),
        in_specs=[
            pl.BlockSpec((BR, 128), lambda i: (i, 0)),
            pl.BlockSpec((128, D1), lambda i: (0, 0)),
            pl.BlockSpec((BR, 1), lambda i: (i, 0)),
            pl.BlockSpec((BR, 1), lambda i: (i, 0)),
        ],
        out_specs=[
            pl.BlockSpec((BR, D1), lambda i: (i, 0)),
            pl.BlockSpec((BR, 1), lambda i: (i, 0)),
        ],
        out_shape=[
            jax.ShapeDtypeStruct((N, D1), jnp.float32),
            jax.ShapeDtypeStruct((N, 1), jnp.float32),
        ],
    )(x, w1p, d0, d1)


def _mid_body(acc_ref, y1_ref, dinv_ref, b1_ref, w2_ref, y2_ref):
    dinv = dinv_ref[...]
    pre = dinv * (acc_ref[0] + acc_ref[1] + y1_ref[...]) + b1_ref[...]
    h = jnp.maximum(pre, 0.0)
    y2_ref[...] = dinv * jnp.dot(h, w2_ref[...], preferred_element_type=jnp.float32)


def _mid_call(acc1, y1, dinv, b1p, w2p):
    return pl.pallas_call(
        _mid_body,
        grid=(N // BR,),
        in_specs=[
            pl.BlockSpec((NC, BR, D1), lambda i: (0, i, 0)),
            pl.BlockSpec((BR, D1), lambda i: (i, 0)),
            pl.BlockSpec((BR, 1), lambda i: (i, 0)),
            pl.BlockSpec((1, D1), lambda i: (0, 0)),
            pl.BlockSpec((D1, D2), lambda i: (0, 0)),
        ],
        out_specs=pl.BlockSpec((BR, D2), lambda i: (i, 0)),
        out_shape=jax.ShapeDtypeStruct((N, D2), jnp.float32),
    )(acc1, y1, dinv, b1p, w2p)


def _fin_body(acc_ref, y2_ref, dinv_ref, b2_ref, z_ref):
    z = dinv_ref[...] * (acc_ref[0] + acc_ref[1] + y2_ref[...]) + b2_ref[...]
    z_ref[...] = z[:, :10]


def _fin_call(acc2, y2, dinv, b2p):
    return pl.pallas_call(
        _fin_body,
        grid=(N // BR,),
        in_specs=[
            pl.BlockSpec((NC, BR, D2), lambda i: (0, i, 0)),
            pl.BlockSpec((BR, D2), lambda i: (i, 0)),
            pl.BlockSpec((BR, 1), lambda i: (i, 0)),
            pl.BlockSpec((1, D2), lambda i: (0, 0)),
        ],
        out_specs=pl.BlockSpec((BR, 10), lambda i: (i, 0)),
        out_shape=jax.ShapeDtypeStruct((N, 10), jnp.float32),
    )(acc2, y2, dinv, b2p)


def kernel(x, edge_index, W1, b1, W2, b2):
    src3 = edge_index[0].reshape(NW, NCH, K)
    dst3 = edge_index[1].reshape(NW, NCH, K)
    w1p = jnp.pad(W1, ((0, 0), (0, D1 - W1.shape[1])))
    b1p = jnp.pad(b1, (0, D1 - b1.shape[0])).reshape(1, D1)
    w2p = jnp.pad(W2, ((0, D1 - W2.shape[0]), (0, D2 - W2.shape[1])))
    b2p = jnp.pad(b2, (0, D2 - b2.shape[0])).reshape(1, D2)

    degp = _deg_call(dst3)
    d0 = degp[0, :N][:, None]
    d1 = degp[1, :N][:, None]
    y1, dinv = _mm1_call(x, w1p, d0, d1)
    acc1 = _agg_call(y1, src3, dst3, D1)
    y2 = _mid_call(acc1, y1, dinv, b1p, w2p)
    acc2 = _agg_call(y2, src3, dst3, D2)
    return _fin_call(acc2, y2, dinv, b2p)


# speedup vs baseline: 1.1404x; 1.1404x over previous
import functools

import jax
import jax.numpy as jnp
import numpy as np
from jax import lax
from jax.experimental import pallas as pl
from jax.experimental.pallas import tpu as pltpu
from jax.experimental.pallas import tpu_sc as plsc

N = 10000
E = 320000
NC = 2
NS = 16
NW = NC * NS
K = 128
NCH = 80
EP = NW * NCH * K
NBUF = 8
GA = 4
D1 = 64
D2 = 16

_mesh = plsc.VectorSubcoreMesh(core_axis_name="c", subcore_axis_name="s")

NP = 10240
_DUM_SRC = np.arange(EP - E, dtype=np.int32) % N
_DUM_DST = N + np.arange(EP - E, dtype=np.int32) % (NP - N)


def _deg_call(dst3):
    @functools.partial(
        pl.kernel,
        out_type=jax.ShapeDtypeStruct((NC, NP), jnp.float32),
        mesh=_mesh,
        scratch_types=[
            pltpu.VMEM((NCH, K), jnp.int32),
            pltpu.VMEM((K,), jnp.float32),
            pltpu.VMEM((640,), jnp.float32),
            pltpu.VMEM_SHARED((NP,), jnp.float32),
            pltpu.SemaphoreType.DMA,
        ],
        compiler_params=pltpu.CompilerParams(use_tc_tiling_on_sc=False),
    )
    def k(dst_hbm, out_hbm, idx_v, ones_v, blk_v, acc_s, sem):
        c = lax.axis_index("c")
        s = lax.axis_index("s")
        wid = c * NS + s
        one16 = jnp.ones((16,), jnp.float32)
        zero16 = jnp.zeros((16,), jnp.float32)
        for i in range(K // 16):
            ones_v[pl.ds(i * 16, 16)] = one16
        for i in range(640 // 16):
            blk_v[pl.ds(i * 16, 16)] = zero16
        pltpu.sync_copy(blk_v, acc_s.at[pl.ds(s * 640, 640)])
        plsc.subcore_barrier()
        pltpu.sync_copy(dst_hbm.at[wid], idx_v)

        W = 8

        def body(j, carry):
            pltpu.async_copy(ones_v, acc_s.at[idx_v.at[j]], sem, add=True)

            @pl.when(j >= W)
            def _():
                pltpu.make_async_copy(ones_v, acc_s.at[idx_v.at[j - W]], sem).wait()

            return carry

        lax.fori_loop(0, NCH, body, 0)
        for t in range(W):
            pltpu.make_async_copy(ones_v, acc_s.at[idx_v.at[NCH - W + t]], sem).wait()
        plsc.subcore_barrier()
        pltpu.sync_copy(acc_s.at[pl.ds(s * 640, 640)], blk_v)
        pltpu.sync_copy(blk_v, out_hbm.at[c, pl.ds(s * 640, 640)])

    return k(dst3)


def _agg_call(y, src3, dst3, dp):
    @functools.partial(
        pl.kernel,
        out_type=jax.ShapeDtypeStruct((NC, NP, 128), jnp.float32),
        mesh=_mesh,
        scratch_types=[
            pltpu.VMEM((NCH, K), jnp.int32),
            pltpu.VMEM((NCH, K), jnp.int32),
            [pltpu.VMEM((K, dp), jnp.float32) for _ in range(NBUF)],
            pltpu.VMEM_SHARED((NP, dp), jnp.float32),
            [pltpu.SemaphoreType.DMA for _ in range(NBUF)],
            [pltpu.SemaphoreType.DMA for _ in range(NBUF)],
        ],
        compiler_params=pltpu.CompilerParams(use_tc_tiling_on_sc=False),
    )
    def k(y_hbm, src_hbm, dst_hbm, out_hbm, si_v, di_v, rows, acc_s, gsem, ssem):
        blk_v = rows[0]
        c = lax.axis_index("c")
        s = lax.axis_index("s")
        wid = c * NS + s
        zero16 = jnp.zeros((16,), jnp.float32)

        def zbody(j, carry):
            for t in range(dp // 16):
                blk_v[j, pl.ds(t * 16, 16)] = zero16
            return carry

        lax.fori_loop(0, K, zbody, 0)
        r0 = s * 640
        for t in range(640 // K):
            pltpu.sync_copy(blk_v, acc_s.at[pl.ds(r0 + t * K, K)])
        plsc.subcore_barrier()

        pltpu.sync_copy(src_hbm.at[wid], si_v)
        pltpu.sync_copy(dst_hbm.at[wid], di_v)

        def gstart(j, b):
            pltpu.async_copy(y_hbm.at[si_v.at[j]], rows[b], gsem[b])

        def gwait(j, b):
            pltpu.make_async_copy(y_hbm.at[si_v.at[j]], rows[b], gsem[b]).wait()

        def sstart(j, b):
            pltpu.async_copy(rows[b], acc_s.at[di_v.at[j]], ssem[b], add=True)

        def swait(j, b):
            pltpu.make_async_copy(rows[b], acc_s.at[di_v.at[j]], ssem[b]).wait()

        NR = NCH // NBUF
        for b in range(GA):
            gstart(b, b)

        def round_body(g, carry):
            j0 = g * NBUF
            for b in range(NBUF):
                j = j0 + b
                bp = (b + GA) % NBUF
                if b < NBUF - GA:
                    @pl.when(g >= 1)
                    def _():
                        swait(j + GA - NBUF, bp)

                    gstart(j + GA, bp)
                else:
                    @pl.when(g < NR - 1)
                    def _():
                        swait(j + GA - NBUF, bp)
                        gstart(j + GA, bp)

                gwait(j, b)
                sstart(j, b)
            return carry

        lax.fori_loop(0, NR, round_body, 0)
        for j in range(NCH - NBUF, NCH):
            swait(j, j % NBUF)
        plsc.subcore_barrier()

        for t in range(640 // K):
            pltpu.sync_copy(acc_s.at[pl.ds(r0 + t * K, K)], blk_v)
            pltpu.sync_copy(blk_v, out_hbm.at[c, pl.ds(r0 + t * K, K), pl.ds(0, dp)])

    return k(y, src3, dst3)


BR = 1000


def _mm1_body(x_ref, w_ref, d0_ref, d1_ref, y_ref, dinv_ref):
    dinv = lax.rsqrt(d0_ref[...] + d1_ref[...] + 1.0)
    xw = jnp.dot(x_ref[...], w_ref[...], preferred_element_type=jnp.float32)
    y_ref[...] = xw * dinv
    dinv_ref[...] = dinv


def _mm1_call(x, w1p, d0, d1):
    return pl.pallas_call(
        _mm1_body,
        grid=(N // BR,),
        in_specs=[
            pl.BlockSpec((BR, 128), lambda i: (i, 0)),
            pl.BlockSpec((128, D1), lambda i: (0, 0)),
            pl.BlockSpec((BR, 1), lambda i: (i, 0)),
            pl.BlockSpec((BR, 1), lambda i: (i, 0)),
        ],
        out_specs=[
            pl.BlockSpec((BR, D1), lambda i: (i, 0)),
            pl.BlockSpec((BR, 1), lambda i: (i, 0)),
        ],
        out_shape=[
            jax.ShapeDtypeStruct((N, D1), jnp.float32),
            jax.ShapeDtypeStruct((N, 1), jnp.float32),
        ],
    )(x, w1p, d0, d1)


def _mid_body(acc_ref, y1_ref, dinv_ref, b1_ref, w2_ref, y2_ref):
    dinv = dinv_ref[...]
    acc = acc_ref[0][:, :D1] + acc_ref[1][:, :D1]
    pre = dinv * (acc + y1_ref[...]) + b1_ref[...]
    h = jnp.maximum(pre, 0.0)
    y2_ref[...] = dinv * jnp.dot(h, w2_ref[...], preferred_element_type=jnp.float32)


def _mid_call(acc1, y1, dinv, b1p, w2p):
    return pl.pallas_call(
        _mid_body,
        grid=(N // BR,),
        in_specs=[
            pl.BlockSpec((NC, BR, 128), lambda i: (0, i, 0)),
            pl.BlockSpec((BR, D1), lambda i: (i, 0)),
            pl.BlockSpec((BR, 1), lambda i: (i, 0)),
            pl.BlockSpec((1, D1), lambda i: (0, 0)),
            pl.BlockSpec((D1, D2), lambda i: (0, 0)),
        ],
        out_specs=pl.BlockSpec((BR, D2), lambda i: (i, 0)),
        out_shape=jax.ShapeDtypeStruct((N, D2), jnp.float32),
    )(acc1, y1, dinv, b1p, w2p)


def _fin_body(acc_ref, y2_ref, dinv_ref, b2_ref, z_ref):
    acc = acc_ref[0][:, :D2] + acc_ref[1][:, :D2]
    z = dinv_ref[...] * (acc + y2_ref[...]) + b2_ref[...]
    z_ref[...] = z[:, :10]


def _fin_call(acc2, y2, dinv, b2p):
    return pl.pallas_call(
        _fin_body,
        grid=(N // BR,),
        in_specs=[
            pl.BlockSpec((NC, BR, 128), lambda i: (0, i, 0)),
            pl.BlockSpec((BR, D2), lambda i: (i, 0)),
            pl.BlockSpec((BR, 1), lambda i: (i, 0)),
            pl.BlockSpec((1, D2), lambda i: (0, 0)),
        ],
        out_specs=pl.BlockSpec((BR, 10), lambda i: (i, 0)),
        out_shape=jax.ShapeDtypeStruct((N, 10), jnp.float32),
    )(acc2, y2, dinv, b2p)


def kernel(x, edge_index, W1, b1, W2, b2):
    src3 = jnp.concatenate([edge_index[0], _DUM_SRC]).reshape(NW, NCH, K)
    dst3 = jnp.concatenate([edge_index[1], _DUM_DST]).reshape(NW, NCH, K)
    w1p = jnp.pad(W1, ((0, 0), (0, D1 - W1.shape[1])))
    b1p = jnp.pad(b1, (0, D1 - b1.shape[0])).reshape(1, D1)
    w2p = jnp.pad(W2, ((0, D1 - W2.shape[0]), (0, D2 - W2.shape[1])))
    b2p = jnp.pad(b2, (0, D2 - b2.shape[0])).reshape(1, D2)

    degp = _deg_call(dst3)
    d0 = degp[0, :N][:, None]
    d1 = degp[1, :N][:, None]
    y1, dinv = _mm1_call(x, w1p, d0, d1)
    acc1 = _agg_call(y1, src3, dst3, D1)
    y2 = _mid_call(acc1, y1, dinv, b1p, w2p)
    acc2 = _agg_call(y2, src3, dst3, D2)
    return _fin_call(acc2, y2, dinv, b2p)


# speedup vs baseline: 1.2553x; 1.1008x over previous
import functools

import jax
import jax.numpy as jnp
import numpy as np
from jax import lax
from jax.experimental import pallas as pl
from jax.experimental.pallas import tpu as pltpu
from jax.experimental.pallas import tpu_sc as plsc

N = 10000
E = 320000
NC = 2
NS = 16
NW = NC * NS
K = 128
NCH = 80
EP = NW * NCH * K
NBUF = 8
GA = 4
D1 = 64
D2 = 16

_mesh = plsc.VectorSubcoreMesh(core_axis_name="c", subcore_axis_name="s")

NP = 10240
_DUM2 = np.stack([
    np.arange(EP - E, dtype=np.int32) % N,
    N + np.arange(EP - E, dtype=np.int32) % (NP - N),
])


def _deg_call(ei4):
    @functools.partial(
        pl.kernel,
        out_type=jax.ShapeDtypeStruct((NC, NP), jnp.float32),
        mesh=_mesh,
        scratch_types=[
            pltpu.VMEM((NCH, K), jnp.int32),
            pltpu.VMEM((K,), jnp.float32),
            pltpu.VMEM((640,), jnp.float32),
            pltpu.VMEM_SHARED((NP,), jnp.float32),
            pltpu.SemaphoreType.DMA,
        ],
        compiler_params=pltpu.CompilerParams(use_tc_tiling_on_sc=False),
    )
    def k(ei_hbm, out_hbm, idx_v, ones_v, blk_v, acc_s, sem):
        c = lax.axis_index("c")
        s = lax.axis_index("s")
        wid = c * NS + s
        one16 = jnp.ones((16,), jnp.float32)
        zero16 = jnp.zeros((16,), jnp.float32)
        for i in range(K // 16):
            ones_v[pl.ds(i * 16, 16)] = one16
        for i in range(640 // 16):
            blk_v[pl.ds(i * 16, 16)] = zero16
        pltpu.sync_copy(blk_v, acc_s.at[pl.ds(s * 640, 640)])
        plsc.subcore_barrier()
        pltpu.sync_copy(ei_hbm.at[1, wid], idx_v)

        W = 8

        def body(j, carry):
            pltpu.async_copy(ones_v, acc_s.at[idx_v.at[j]], sem, add=True)

            @pl.when(j >= W)
            def _():
                pltpu.make_async_copy(ones_v, acc_s.at[idx_v.at[j - W]], sem).wait()

            return carry

        lax.fori_loop(0, NCH, body, 0)
        for t in range(W):
            pltpu.make_async_copy(ones_v, acc_s.at[idx_v.at[NCH - W + t]], sem).wait()
        plsc.subcore_barrier()
        pltpu.sync_copy(acc_s.at[pl.ds(s * 640, 640)], blk_v)
        pltpu.sync_copy(blk_v, out_hbm.at[c, pl.ds(s * 640, 640)])

    return k(ei4)


def _agg_call(y, ei4, dp):
    @functools.partial(
        pl.kernel,
        out_type=jax.ShapeDtypeStruct((NC, NP, 128), jnp.float32),
        mesh=_mesh,
        scratch_types=[
            pltpu.VMEM((NCH, K), jnp.int32),
            pltpu.VMEM((NCH, K), jnp.int32),
            [pltpu.VMEM((K, dp), jnp.float32) for _ in range(NBUF)],
            pltpu.VMEM_SHARED((NP, dp), jnp.float32),
            [pltpu.SemaphoreType.DMA for _ in range(NBUF)],
            [pltpu.SemaphoreType.DMA for _ in range(NBUF)],
        ],
        compiler_params=pltpu.CompilerParams(use_tc_tiling_on_sc=False),
    )
    def k(y_hbm, ei_hbm, out_hbm, si_v, di_v, rows, acc_s, gsem, ssem):
        blk_v = rows[0]
        c = lax.axis_index("c")
        s = lax.axis_index("s")
        wid = c * NS + s
        zero16 = jnp.zeros((16,), jnp.float32)

        def zbody(j, carry):
            for t in range(dp // 16):
                blk_v[j, pl.ds(t * 16, 16)] = zero16
            return carry

        lax.fori_loop(0, K, zbody, 0)
        r0 = s * 640
        for t in range(640 // K):
            pltpu.sync_copy(blk_v, acc_s.at[pl.ds(r0 + t * K, K)])
        plsc.subcore_barrier()

        pltpu.sync_copy(ei_hbm.at[0, wid], si_v)
        pltpu.sync_copy(ei_hbm.at[1, wid], di_v)

        def gstart(j, b):
            pltpu.async_copy(y_hbm.at[si_v.at[j]], rows[b], gsem[b])

        def gwait(j, b):
            pltpu.make_async_copy(y_hbm.at[si_v.at[j]], rows[b], gsem[b]).wait()

        def sstart(j, b):
            pltpu.async_copy(rows[b], acc_s.at[di_v.at[j]], ssem[b], add=True)

        def swait(j, b):
            pltpu.make_async_copy(rows[b], acc_s.at[di_v.at[j]], ssem[b]).wait()

        NR = NCH // NBUF
        for b in range(GA):
            gstart(b, b)

        def round_body(g, carry):
            j0 = g * NBUF
            for b in range(NBUF):
                j = j0 + b
                bp = (b + GA) % NBUF
                if b < NBUF - GA:
                    @pl.when(g >= 1)
                    def _():
                        swait(j + GA - NBUF, bp)

                    gstart(j + GA, bp)
                else:
                    @pl.when(g < NR - 1)
                    def _():
                        swait(j + GA - NBUF, bp)
                        gstart(j + GA, bp)

                gwait(j, b)
                sstart(j, b)
            return carry

        lax.fori_loop(0, NR, round_body, 0)
        for j in range(NCH - NBUF, NCH):
            swait(j, j % NBUF)
        plsc.subcore_barrier()

        for t in range(640 // K):
            pltpu.sync_copy(acc_s.at[pl.ds(r0 + t * K, K)], blk_v)
            pltpu.sync_copy(blk_v, out_hbm.at[c, pl.ds(r0 + t * K, K), pl.ds(0, dp)])

    return k(y, ei4)


BR = 2000


def _mm1_body(x_ref, w_ref, d0_ref, d1_ref, y_ref, dinv_ref):
    dinv = lax.rsqrt(d0_ref[...] + d1_ref[...] + 1.0)
    xw = jnp.dot(x_ref[...], w_ref[...], preferred_element_type=jnp.float32)
    y_ref[...] = xw * dinv
    dinv_ref[...] = dinv


def _mm1_call(x, w1p, d0, d1):
    return pl.pallas_call(
        _mm1_body,
        grid=(N // BR,),
        in_specs=[
            pl.BlockSpec((BR, 128), lambda i: (i, 0)),
            pl.BlockSpec((128, D1), lambda i: (0, 0)),
            pl.BlockSpec((BR, 1), lambda i: (i, 0)),
            pl.BlockSpec((BR, 1), lambda i: (i, 0)),
        ],
        out_specs=[
            pl.BlockSpec((BR, D1), lambda i: (i, 0)),
            pl.BlockSpec((BR, 1), lambda i: (i, 0)),
        ],
        out_shape=[
            jax.ShapeDtypeStruct((N, D1), jnp.float32),
            jax.ShapeDtypeStruct((N, 1), jnp.float32),
        ],
    )(x, w1p, d0, d1)


def _mid_body(acc_ref, y1_ref, dinv_ref, b1_ref, w2_ref, y2_ref):
    dinv = dinv_ref[...]
    acc = acc_ref[0][:, :D1] + acc_ref[1][:, :D1]
    pre = dinv * (acc + y1_ref[...]) + b1_ref[...]
    h = jnp.maximum(pre, 0.0)
    y2_ref[...] = dinv * jnp.dot(h, w2_ref[...], preferred_element_type=jnp.float32)


def _mid_call(acc1, y1, dinv, b1p, w2p):
    return pl.pallas_call(
        _mid_body,
        grid=(N // BR,),
        in_specs=[
            pl.BlockSpec((NC, BR, 128), lambda i: (0, i, 0)),
            pl.BlockSpec((BR, D1), lambda i: (i, 0)),
            pl.BlockSpec((BR, 1), lambda i: (i, 0)),
            pl.BlockSpec((1, D1), lambda i: (0, 0)),
            pl.BlockSpec((D1, D2), lambda i: (0, 0)),
        ],
        out_specs=pl.BlockSpec((BR, D2), lambda i: (i, 0)),
        out_shape=jax.ShapeDtypeStruct((N, D2), jnp.float32),
    )(acc1, y1, dinv, b1p, w2p)


def _fin_body(acc_ref, y2_ref, dinv_ref, b2_ref, z_ref):
    acc = acc_ref[0][:, :D2] + acc_ref[1][:, :D2]
    z = dinv_ref[...] * (acc + y2_ref[...]) + b2_ref[...]
    z_ref[...] = z[:, :10]


def _fin_call(acc2, y2, dinv, b2p):
    return pl.pallas_call(
        _fin_body,
        grid=(N // BR,),
        in_specs=[
            pl.BlockSpec((NC, BR, 128), lambda i: (0, i, 0)),
            pl.BlockSpec((BR, D2), lambda i: (i, 0)),
            pl.BlockSpec((BR, 1), lambda i: (i, 0)),
            pl.BlockSpec((1, D2), lambda i: (0, 0)),
        ],
        out_specs=pl.BlockSpec((BR, 10), lambda i: (i, 0)),
        out_shape=jax.ShapeDtypeStruct((N, 10), jnp.float32),
    )(acc2, y2, dinv, b2p)


def kernel(x, edge_index, W1, b1, W2, b2):
    ei4 = jnp.concatenate([edge_index, _DUM2], axis=1).reshape(2, NW, NCH, K)
    w1p = jnp.pad(W1, ((0, 0), (0, D1 - W1.shape[1])))
    b1p = jnp.pad(b1, (0, D1 - b1.shape[0])).reshape(1, D1)
    w2p = jnp.pad(W2, ((0, D1 - W2.shape[0]), (0, D2 - W2.shape[1])))
    b2p = jnp.pad(b2, (0, D2 - b2.shape[0])).reshape(1, D2)

    degp = _deg_call(ei4)
    d0 = degp[0, :N][:, None]
    d1 = degp[1, :N][:, None]
    y1, dinv = _mm1_call(x, w1p, d0, d1)
    acc1 = _agg_call(y1, ei4, D1)
    y2 = _mid_call(acc1, y1, dinv, b1p, w2p)
    acc2 = _agg_call(y2, ei4, D2)
    return _fin_call(acc2, y2, dinv, b2p)
